# baseline (speedup 1.0000x reference)
import functools

import jax
import jax.numpy as jnp
from jax import lax
from jax.experimental import pallas as pl
from jax.experimental.pallas import tpu as pltpu
from jax.experimental.pallas import tpu_sc as plsc

N_TOKENS = 8192
HIDDEN = 4096
NPOP = 8
BLK = 512
NBLK = N_TOKENS // BLK

NC = 2
NS = 16
NW = NC * NS
CHUNK = N_TOKENS // NW
LANES = 16


def _logits_kernel(x_ref, w_ref, b_ref, hid_ref, logit_ref):
    xb = x_ref[...]
    hid_ref[...] = xb
    w = w_ref[...]
    logits = lax.dot_general(
        xb, w, (((1,), (1,)), ((), ())),
        preferred_element_type=jnp.float32)
    logit_ref[...] = logits + b_ref[...][None, :]


def _route_kernel(logit_hbm, ids_hbm, out_hbm, ids_v, p_v, out_v):
    wid = lax.axis_index("s") * NC + lax.axis_index("c")
    base = wid * CHUNK
    pltpu.sync_copy(ids_hbm.at[pl.ds(base, CHUNK)], ids_v)
    pltpu.sync_copy(logit_hbm.at[pl.ds(base * NPOP, CHUNK * NPOP)], p_v)
    for j in range(CHUNK // LANES):
        rows = lax.iota(jnp.int32, LANES) + j * LANES
        cols = ids_v[pl.ds(j * LANES, LANES)]
        flat = rows * NPOP + cols
        out_v[pl.ds(j * LANES, LANES)] = plsc.load_gather(p_v, [flat])
    pltpu.sync_copy(out_v, out_hbm.at[pl.ds(base, CHUNK)])


def kernel(x, pop_ids, W, b):
    hidden, logits = pl.pallas_call(
        _logits_kernel,
        grid=(NBLK,),
        in_specs=[
            pl.BlockSpec((BLK, HIDDEN), lambda i: (i, 0)),
            pl.BlockSpec((NPOP, HIDDEN), lambda i: (0, 0)),
            pl.BlockSpec((NPOP,), lambda i: (0,)),
        ],
        out_specs=[
            pl.BlockSpec((BLK, HIDDEN), lambda i: (i, 0)),
            pl.BlockSpec((BLK, NPOP), lambda i: (i, 0)),
        ],
        out_shape=[
            jax.ShapeDtypeStruct((N_TOKENS, HIDDEN), jnp.float32),
            jax.ShapeDtypeStruct((N_TOKENS, NPOP), jnp.float32),
        ],
    )(x, W, b)

    route = functools.partial(
        pl.kernel,
        out_type=jax.ShapeDtypeStruct((N_TOKENS,), jnp.float32),
        mesh=plsc.VectorSubcoreMesh(
            core_axis_name="c", subcore_axis_name="s",
            num_cores=NC, num_subcores=NS),
        scratch_types=[
            pltpu.VMEM((CHUNK,), jnp.int32),
            pltpu.VMEM((CHUNK * NPOP,), jnp.float32),
            pltpu.VMEM((CHUNK,), jnp.float32),
        ],
        compiler_params=pltpu.CompilerParams(needs_layout_passes=False),
    )(_route_kernel)
    values = route(logits.reshape(N_TOKENS * NPOP), pop_ids)
    return (hidden, values.reshape(N_TOKENS, 1))
